# baseline (speedup 1.0000x reference)
import functools

import jax
import jax.numpy as jnp
from jax import lax
from jax.experimental import pallas as pl
from jax.experimental.pallas import tpu as pltpu
from jax.experimental.pallas import tpu_sc as plsc

LANES = 16


@functools.lru_cache(maxsize=None)
def _make_emb_kernel(n_rows, vocab, d_model, seq_len):
    info = plsc.get_sparse_core_info()
    nc, ns = info.num_cores, info.num_subcores
    nw = nc * ns
    assert n_rows % nw == 0
    b_per_w = n_rows // nw
    ch = 64
    assert b_per_w % ch == 0
    n_chunks = b_per_w // ch
    assert seq_len % ch == 0
    assert d_model % LANES == 0
    cols = d_model // LANES

    mesh = plsc.VectorSubcoreMesh(core_axis_name="c", subcore_axis_name="s")

    @functools.partial(
        pl.kernel,
        mesh=mesh,
        out_type=jax.ShapeDtypeStruct((n_rows, d_model), jnp.float32),
        scratch_types=[
            pltpu.VMEM((ch,), jnp.int32),
            pltpu.VMEM((ch, d_model), jnp.float32),
            pltpu.VMEM((ch, d_model), jnp.float32),
            pltpu.SemaphoreType.DMA,
        ],
    )
    def emb(idx_hbm, tok_hbm, pos_hbm, out_hbm, idx_v, tok_v, acc_v, sem):
        wid = lax.axis_index("s") * nc + lax.axis_index("c")
        base = wid * b_per_w

        def chunk_body(g, carry):
            row0 = base + g * ch
            pos0 = lax.rem(row0, seq_len)
            pltpu.sync_copy(idx_hbm.at[pl.ds(row0, ch)], idx_v)
            gather = pltpu.async_copy(tok_hbm.at[idx_v], tok_v, sem)
            pltpu.sync_copy(pos_hbm.at[pl.ds(pos0, ch)], acc_v)
            gather.wait()

            def row_body(r, c2):
                def col_body(c, r2):
                    s = c * LANES
                    plsc.addupdate(
                        acc_v.at[r2, pl.ds(s, LANES)],
                        tok_v[r2, pl.ds(s, LANES)],
                    )
                    return r2

                return lax.fori_loop(0, cols, col_body, r)

            lax.fori_loop(0, ch, row_body, 0)
            pltpu.sync_copy(acc_v, out_hbm.at[pl.ds(row0, ch)])
            return carry

        lax.fori_loop(0, n_chunks, chunk_body, 0)

    return emb


def kernel(x, token_table, pos_table):
    b, s = x.shape
    v, d = token_table.shape
    idx = x.reshape(b * s).astype(jnp.int32)
    emb = _make_emb_kernel(b * s, v, d, pos_table.shape[0])
    out = emb(idx, token_table, pos_table)
    return out.reshape(b, s, d)


# speedup vs baseline: 2.1744x; 2.1744x over previous
import functools

import jax
import jax.numpy as jnp
from jax import lax
from jax.experimental import pallas as pl
from jax.experimental.pallas import tpu as pltpu
from jax.experimental.pallas import tpu_sc as plsc

LANES = 16
NBUF = 3


@functools.lru_cache(maxsize=None)
def _make_emb_kernel(batch, seq, vocab, d_model):
    info = plsc.get_sparse_core_info()
    nc, ns = info.num_cores, info.num_subcores
    nw = nc * ns
    assert seq % nw == 0
    s_per_w = seq // nw
    ch = 32
    n_halves = s_per_w // ch
    n_chunks = n_halves * batch
    assert d_model % LANES == 0
    cols = d_model // LANES

    mesh = plsc.VectorSubcoreMesh(core_axis_name="c", subcore_axis_name="s")

    @functools.partial(
        pl.kernel,
        mesh=mesh,
        out_type=jax.ShapeDtypeStruct((batch * seq, d_model), jnp.float32),
        scratch_types=(
            [pltpu.VMEM((ch,), jnp.int32) for _ in range(NBUF)]
            + [pltpu.VMEM((ch, d_model), jnp.float32) for _ in range(NBUF)]
            + [pltpu.VMEM((ch, d_model), jnp.float32) for _ in range(n_halves)]
            + [pltpu.SemaphoreType.DMA for _ in range(2 * NBUF + n_halves)]
        ),
    )
    def emb(idx_hbm, tok_hbm, pos_hbm, out_hbm, *refs):
        idx_v = refs[0:NBUF]
        tok_v = refs[NBUF:2 * NBUF]
        pos_v = refs[2 * NBUF:2 * NBUF + n_halves]
        gsem = refs[2 * NBUF + n_halves:3 * NBUF + n_halves]
        wsem = refs[3 * NBUF + n_halves:4 * NBUF + n_halves]
        psem = refs[4 * NBUF + n_halves:]

        wid = lax.axis_index("s") * nc + lax.axis_index("c")
        s0 = wid * s_per_w

        def row0_of(g):
            h, k = divmod(g, batch)
            return k * seq + s0 + h * ch

        def issue_gather(g):
            b = g % NBUF
            row0 = row0_of(g)
            pltpu.sync_copy(idx_hbm.at[pl.ds(row0, ch)], idx_v[b])
            return pltpu.async_copy(tok_hbm.at[idx_v[b]], tok_v[b], gsem[b])

        def issue_out(g):
            b = g % NBUF
            return pltpu.async_copy(
                tok_v[b], out_hbm.at[pl.ds(row0_of(g), ch)], wsem[b]
            )

        pos_cp = [
            pltpu.async_copy(
                pos_hbm.at[pl.ds(s0 + h * ch, ch)], pos_v[h], psem[h]
            )
            for h in range(n_halves)
        ]

        def add_chunk(g):
            b = g % NBUF
            h = g // batch

            def row_body(r, carry):
                for c in range(cols):
                    s = c * LANES
                    plsc.addupdate(
                        tok_v[b].at[r, pl.ds(s, LANES)],
                        pos_v[h][r, pl.ds(s, LANES)],
                    )
                return carry

            lax.fori_loop(0, ch, row_body, 0)

        gather_cp = {g: issue_gather(g) for g in range(min(2, n_chunks))}
        for h in range(n_halves):
            pos_cp[h].wait()
        out_cp = {}
        for g in range(n_chunks):
            gather_cp[g].wait()
            add_chunk(g)
            out_cp[g] = issue_out(g)
            if g + 2 < n_chunks:
                if g - 1 >= 0:
                    out_cp[g - 1].wait()
                gather_cp[g + 2] = issue_gather(g + 2)
        for g in range(n_chunks - 3, n_chunks):
            out_cp[g].wait()

    return emb


def kernel(x, token_table, pos_table):
    b, s = x.shape
    v, d = token_table.shape
    idx = x.reshape(b * s).astype(jnp.int32)
    emb = _make_emb_kernel(b, s, v, d)
    out = emb(idx, token_table, pos_table)
    return out.reshape(b, s, d)
